# baseline (speedup 1.0000x reference)
import functools

import jax
import jax.numpy as jnp
from jax import lax
from jax.experimental import pallas as pl
from jax.experimental.pallas import tpu as pltpu
from jax.experimental.pallas import tpu_sc as plsc

VOCAB = 1000000
SEQ = 200
EMBED = 64
BATCH = 4096

_NC = 2
_NS = 16
_NW = _NC * _NS

_B_PER_W = BATCH // _NW
_CHUNK_B = 2
_CHUNK = _CHUNK_B * SEQ
_CHUNKS = _B_PER_W // _CHUNK_B
_LANES = 16
_VPR = EMBED // _LANES
_URF = 4

_PIECES = [(0, 128), (128, 72)]


@functools.partial(
    pl.kernel,
    mesh=plsc.VectorSubcoreMesh(core_axis_name="c", subcore_axis_name="s"),
    compiler_params=pltpu.CompilerParams(use_tc_tiling_on_sc=False),
    out_type=jax.ShapeDtypeStruct((BATCH, SEQ, EMBED), jnp.float32),
    scratch_types=[
        pltpu.VMEM((_CHUNK,), jnp.int32),
        pltpu.VMEM((_CHUNK,), jnp.int32),
        pltpu.VMEM((_CHUNK_B, SEQ, EMBED), jnp.float32),
        pltpu.VMEM((_CHUNK_B, SEQ, EMBED), jnp.float32),
        pltpu.VMEM((SEQ, EMBED), jnp.float32),
        pltpu.SemaphoreType.DMA,
        pltpu.SemaphoreType.DMA,
        pltpu.SemaphoreType.DMA,
        pltpu.SemaphoreType.DMA,
        pltpu.SemaphoreType.DMA,
        pltpu.SemaphoreType.DMA,
    ],
)
def _emb_kernel(idx_hbm, tok_hbm, pos_hbm, out_hbm,
                idx0, idx1, rows0, rows1, pos_v,
                isem0, isem1, gsem0, gsem1, ssem0, ssem1):
    wid = lax.axis_index("s") * _NC + lax.axis_index("c")
    bbase = wid * _B_PER_W
    pltpu.sync_copy(pos_hbm, pos_v)

    idx = (idx0, idx1)
    rows = (rows0, rows1)
    isem = (isem0, isem1)
    gsem = (gsem0, gsem1)
    ssem = (ssem0, ssem1)

    def fire_idx(c, buf):
        pltpu.async_copy(idx_hbm.at[pl.ds((bbase + c * _CHUNK_B) * SEQ, _CHUNK)],
                         idx[buf], isem[buf])

    def wait_idx(buf):
        pltpu.make_async_copy(idx_hbm.at[pl.ds(0, _CHUNK)],
                              idx[buf], isem[buf]).wait()

    def fire_gathers(buf):
        for t in range(_CHUNK_B):
            for o, n in _PIECES:
                pltpu.async_copy(
                    tok_hbm.at[idx[buf].at[pl.ds(t * SEQ + o, n)]],
                    rows[buf].at[t, pl.ds(o, n)],
                    gsem[buf])

    def wait_gathers(buf):
        for t in range(_CHUNK_B):
            for o, n in _PIECES:
                pltpu.make_async_copy(
                    tok_hbm.at[idx[buf].at[pl.ds(t * SEQ + o, n)]],
                    rows[buf].at[t, pl.ds(o, n)],
                    gsem[buf]).wait()

    def fire_scatter(c, buf):
        pltpu.async_copy(rows[buf],
                         out_hbm.at[pl.ds(bbase + c * _CHUNK_B, _CHUNK_B)],
                         ssem[buf])

    def wait_scatter(buf):
        pltpu.make_async_copy(rows[buf], out_hbm.at[pl.ds(0, _CHUNK_B)],
                              ssem[buf]).wait()

    def add_pos(buf):
        r = rows[buf]

        def grp(g, carry):
            for ss in range(_URF):
                s = g * _URF + ss
                for j in range(_VPR):
                    sl = pl.ds(j * _LANES, _LANES)
                    pv = pos_v[s, sl]
                    for t in range(_CHUNK_B):
                        r[t, s, sl] = r[t, s, sl] + pv
            return carry

        lax.fori_loop(0, SEQ // _URF, grp, 0)

    def step(c, buf, fire_next_idx=True, wait_sc=True):
        obuf = 1 - buf
        wait_gathers(obuf)
        wait_idx(buf)
        if wait_sc:
            wait_scatter(buf)
        fire_gathers(buf)
        if fire_next_idx:
            fire_idx(c + 1, obuf)
        add_pos(obuf)
        fire_scatter(c - 1, obuf)

    fire_idx(0, 0)
    wait_idx(0)
    fire_idx(1, 1)
    fire_gathers(0)
    step(1, 1, wait_sc=False)

    def super_body(i, carry):
        step(2 * i, 0)
        step(2 * i + 1, 1)
        return carry

    lax.fori_loop(1, _CHUNKS // 2 - 1, super_body, 0)

    step(_CHUNKS - 2, 0)
    step(_CHUNKS - 1, 1, fire_next_idx=False)
    wait_gathers(1)
    add_pos(1)
    fire_scatter(_CHUNKS - 1, 1)
    wait_scatter(0)
    wait_scatter(1)


def kernel(inputs, token_table, pos_table):
    idx = jnp.minimum(jnp.maximum(inputs, 0), VOCAB - 1).reshape(-1)
    return _emb_kernel(idx, token_table, pos_table)
